# baseline (speedup 1.0000x reference)
import functools

import jax
import jax.numpy as jnp
from jax import lax
from jax.experimental import pallas as pl
from jax.experimental.pallas import tpu as pltpu
from jax.experimental.pallas import tpu_sc as plsc

N = 10000
E = 160000
F_X = 256
F_E = 16
F_U = 64
H = 128
G = 16

NC = 2
NS = 16
L = 16
NW = NC * NS

C = 128
NCHUNK = E // C
MAXCH = (NCHUNK + NW - 1) // NW
NPAD = 10240
ROWS_PER_SUB = NPAD // NS

TB = 1000
TGRID = N // TB


def _proj_body(x_ref, wk_ref, bk_ref, wq_ref, bq_ref,
               batch_ref, u_ref, wx1_ref, wx3_ref, bx_ref,
               k_ref, q_ref, p_ref):
    xb = x_ref[...].astype(jnp.bfloat16)
    kv = jnp.dot(xb, wk_ref[...].astype(jnp.bfloat16),
                 preferred_element_type=jnp.float32) + bk_ref[...]
    qv = jnp.dot(xb, wq_ref[...].astype(jnp.bfloat16),
                 preferred_element_type=jnp.float32) + bq_ref[...]
    k_ref[...] = kv
    q_ref[...] = qv
    b = batch_ref[0, 0, :]
    onehot = (b[:, None] == lax.broadcasted_iota(jnp.int32, (1, G), 1)
              ).astype(jnp.float32)
    uw = jnp.dot(u_ref[...], wx3_ref[...], preferred_element_type=jnp.float32)
    o = jnp.dot(xb, wx1_ref[...].astype(jnp.bfloat16),
                preferred_element_type=jnp.float32)
    o = o + jnp.dot(onehot, uw, preferred_element_type=jnp.float32)
    p_ref[...] = o + bx_ref[...]


def _project(x, Wk, bk, Wq, bq, batch3d, u, Wx1, Wx3, bx):
    return pl.pallas_call(
        _proj_body,
        grid=(TGRID,),
        in_specs=[
            pl.BlockSpec((TB, F_X), lambda i: (i, 0)),
            pl.BlockSpec((F_X, H), lambda i: (0, 0)),
            pl.BlockSpec((1, H), lambda i: (0, 0)),
            pl.BlockSpec((F_X, H), lambda i: (0, 0)),
            pl.BlockSpec((1, H), lambda i: (0, 0)),
            pl.BlockSpec((1, 1, TB), lambda i: (i, 0, 0)),
            pl.BlockSpec((G, F_U), lambda i: (0, 0)),
            pl.BlockSpec((F_X, F_X), lambda i: (0, 0)),
            pl.BlockSpec((F_U, F_X), lambda i: (0, 0)),
            pl.BlockSpec((1, F_X), lambda i: (0, 0)),
        ],
        out_specs=[
            pl.BlockSpec((TB, H), lambda i: (i, 0)),
            pl.BlockSpec((TB, H), lambda i: (i, 0)),
            pl.BlockSpec((TB, F_X), lambda i: (i, 0)),
        ],
        out_shape=[
            jax.ShapeDtypeStruct((N, H), jnp.float32),
            jax.ShapeDtypeStruct((N, H), jnp.float32),
            jax.ShapeDtypeStruct((N, F_X), jnp.float32),
        ],
    )(x, Wk, bk.reshape(1, H), Wq, bq.reshape(1, H),
      batch3d, u, Wx1, Wx3, bx.reshape(1, F_X))


_GDN = lax.GatherDimensionNumbers(
    offset_dims=(), collapsed_slice_dims=(0,), start_index_map=(0,))


def _lane_shuffle(v, idx):
    return lax.gather(v, idx[:, None], dimension_numbers=_GDN,
                      slice_sizes=(1,),
                      mode=lax.GatherScatterMode.PROMISE_IN_BOUNDS)


def _edge_body(k_hbm, q_hbm, src_hbm, dst_hbm, e_hbm, out_hbm,
               sidx, didx, krows, qrows, erows, aerows, zbuf, acc,
               sem_k, sem_q, sem_e, sem_s):
    c = lax.axis_index("c")
    s = lax.axis_index("s")
    wid = s * NC + c
    start = wid * NCHUNK // NW
    end = (wid + 1) * NCHUNK // NW

    def _zrow(i, carry):
        zbuf[i, :] = jnp.zeros((L,), jnp.float32)
        return carry
    lax.fori_loop(0, ROWS_PER_SUB, _zrow, 0)
    pltpu.sync_copy(zbuf, acc.at[pl.ds(s * ROWS_PER_SUB, ROWS_PER_SUB)])

    pltpu.sync_copy(src_hbm.at[pl.ds(start, MAXCH)], sidx)
    pltpu.sync_copy(dst_hbm.at[pl.ds(start, MAXCH)], didx)
    plsc.subcore_barrier()

    EROWS = C * F_E // 128

    def _fire(v, slot):
        pltpu.async_copy(k_hbm.at[sidx.at[v - start]], krows.at[slot], sem_k)
        pltpu.async_copy(q_hbm.at[didx.at[v - start]], qrows.at[slot], sem_q)
        pltpu.async_copy(e_hbm.at[pl.ds(v * EROWS, EROWS)], erows.at[slot],
                         sem_e)

    def _await(v, slot):
        pltpu.make_async_copy(k_hbm.at[sidx.at[v - start]], krows.at[slot],
                              sem_k).wait()
        pltpu.make_async_copy(q_hbm.at[didx.at[v - start]], qrows.at[slot],
                              sem_q).wait()
        pltpu.make_async_copy(e_hbm.at[pl.ds(v * EROWS, EROWS)],
                              erows.at[slot], sem_e).wait()

    lanes = lax.iota(jnp.int32, L)
    msk8 = lanes < 8
    msk4 = (lanes & 4) == 0
    msk2 = (lanes & 2) == 0
    msk1 = (lanes & 1) == 0
    REV = (0, 8, 4, 12, 2, 10, 6, 14, 1, 9, 5, 13, 3, 11, 7, 15)

    def _fold(v, k):
        return v + _lane_shuffle(v, lanes ^ k)

    def _compute(v, slot):
        _await(v, slot)
        kr = krows.at[slot]
        qr = qrows.at[slot]
        er = erows.at[slot]
        ar = aerows.at[slot]

        def _dot(i):
            d = kr[i, 0:L] * qr[i, 0:L]
            for g in range(1, H // L):
                d = d + kr[i, g * L:(g + 1) * L] * qr[i, g * L:(g + 1) * L]
            return d

        def _blk16(b, ecarry):
            base = b * 16
            m = []
            for p in range(8):
                d0 = _dot(base + 2 * p)
                d1 = _dot(base + 2 * p + 1)
                m.append(jnp.where(msk8, _fold(d0, 8), _fold(d1, 8)))
            n = [jnp.where(msk4, _fold(m[2 * q], 4), _fold(m[2 * q + 1], 4))
                 for q in range(4)]
            r0 = jnp.where(msk2, _fold(n[0], 2), _fold(n[1], 2))
            r1 = jnp.where(msk2, _fold(n[2], 2), _fold(n[3], 2))
            f = jnp.where(msk1, _fold(r0, 1), _fold(r1, 1))
            a = 1.0 / (1.0 + jnp.exp(-f))
            for j in range(16):
                i = base + j
                aj = _lane_shuffle(a, jnp.full((L,), REV[j], jnp.int32))
                ar[i, :] = aj * er[i // 8, pl.ds((i % 8) * L, L)]
            return ecarry
        lax.fori_loop(0, C // 16, _blk16, 0)
        pltpu.async_copy(ar, acc.at[didx.at[v - start]], sem_s,
                         add=True).wait()

    _fire(start, 0)
    _fire(start + 1, 1)

    def _pair(i2, carry):
        v0 = start + 2 * i2
        v1 = v0 + 1

        @pl.when(v0 < end)
        def _():
            _compute(v0, 0)

            @pl.when(v0 + 2 < end)
            def _():
                _fire(v0 + 2, 0)

        @pl.when(v1 < end)
        def _():
            _compute(v1, 1)

            @pl.when(v1 + 2 < end)
            def _():
                _fire(v1 + 2, 1)
        return carry
    lax.fori_loop(0, MAXCH // 2, _pair, 0)

    plsc.subcore_barrier()
    pltpu.sync_copy(acc.at[pl.ds(s * ROWS_PER_SUB, ROWS_PER_SUB)],
                    out_hbm.at[c, pl.ds(s * ROWS_PER_SUB, ROWS_PER_SUB)])


def _edge_aggregate(k_all, q_all, src, dst, e):
    mesh = plsc.VectorSubcoreMesh(core_axis_name="c", subcore_axis_name="s")
    fn = functools.partial(
        pl.kernel,
        mesh=mesh,
        compiler_params=pltpu.CompilerParams(use_tc_tiling_on_sc=False),
        out_type=jax.ShapeDtypeStruct((NC, NPAD, F_E), jnp.float32),
        scratch_types=[
            pltpu.VMEM((MAXCH, C), jnp.int32),
            pltpu.VMEM((MAXCH, C), jnp.int32),
            pltpu.VMEM((2, C, H), jnp.float32),
            pltpu.VMEM((2, C, H), jnp.float32),
            pltpu.VMEM((2, C * F_E // 128, 128), jnp.float32),
            pltpu.VMEM((2, C, F_E), jnp.float32),
            pltpu.VMEM((ROWS_PER_SUB, F_E), jnp.float32),
            pltpu.VMEM_SHARED((NPAD, F_E), jnp.float32),
            pltpu.SemaphoreType.DMA,
            pltpu.SemaphoreType.DMA,
            pltpu.SemaphoreType.DMA,
            pltpu.SemaphoreType.DMA,
        ],
    )(_edge_body)
    return fn(k_all, q_all, src.reshape(NCHUNK, C), dst.reshape(NCHUNK, C),
              e.reshape(E * F_E // 128, 128))


def _final_body(p_ref, a0_ref, a1_ref, wx2_ref, o_ref):
    aggb = a0_ref[...] + a1_ref[...]
    o_ref[...] = p_ref[...] + jnp.dot(aggb, wx2_ref[...],
                                      preferred_element_type=jnp.float32)


def _final(part, agg0, agg1, Wx2):
    return pl.pallas_call(
        _final_body,
        grid=(TGRID,),
        in_specs=[
            pl.BlockSpec((TB, F_X), lambda i: (i, 0)),
            pl.BlockSpec((TB, F_E), lambda i: (i, 0)),
            pl.BlockSpec((TB, F_E), lambda i: (i, 0)),
            pl.BlockSpec((F_E, F_X), lambda i: (0, 0)),
        ],
        out_specs=pl.BlockSpec((TB, F_X), lambda i: (i, 0)),
        out_shape=jax.ShapeDtypeStruct((N, F_X), jnp.float32),
    )(part, agg0, agg1, Wx2)


def kernel(x, edge_index, e, u, batch, Wk, bk, Wq, bq, Wx, bx):
    src = edge_index[0].astype(jnp.int32)
    dst = edge_index[1].astype(jnp.int32)
    batch3d = batch.astype(jnp.int32).reshape(TGRID, 1, TB)

    k_all, q_all, part = _project(x, Wk, bk, Wq, bq, batch3d, u,
                                  Wx[:F_X], Wx[F_X + F_E:], bx)
    agg2 = _edge_aggregate(k_all, q_all, src, dst, e)
    return _final(part, agg2[0, :N], agg2[1, :N], Wx[F_X:F_X + F_E])


# speedup vs baseline: 1.0393x; 1.0393x over previous
import functools

import jax
import jax.numpy as jnp
from jax import lax
from jax.experimental import pallas as pl
from jax.experimental.pallas import tpu as pltpu
from jax.experimental.pallas import tpu_sc as plsc

N = 10000
E = 160000
F_X = 256
F_E = 16
F_U = 64
H = 128
G = 16

NC = 2
NS = 16
L = 16
NW = NC * NS

C = 128
NCHUNK = E // C
MAXCH = (NCHUNK + NW - 1) // NW
NPAD = 10240
ROWS_PER_SUB = NPAD // NS

TB = 1000
TGRID = N // TB


def _proj_body(x_ref, wk_ref, bk_ref, wq_ref, bq_ref, k_ref, q_ref):
    xb = x_ref[...]
    kv = jnp.dot(xb, wk_ref[...], preferred_element_type=jnp.float32) + bk_ref[...]
    qv = jnp.dot(xb, wq_ref[...], preferred_element_type=jnp.float32) + bq_ref[...]
    k_ref[...] = kv
    q_ref[...] = qv


def _project(x, Wk, bk, Wq, bq):
    return pl.pallas_call(
        _proj_body,
        grid=(TGRID,),
        in_specs=[
            pl.BlockSpec((TB, F_X), lambda i: (i, 0)),
            pl.BlockSpec((F_X, H), lambda i: (0, 0)),
            pl.BlockSpec((1, H), lambda i: (0, 0)),
            pl.BlockSpec((F_X, H), lambda i: (0, 0)),
            pl.BlockSpec((1, H), lambda i: (0, 0)),
        ],
        out_specs=[
            pl.BlockSpec((TB, H), lambda i: (i, 0)),
            pl.BlockSpec((TB, H), lambda i: (i, 0)),
        ],
        out_shape=[
            jax.ShapeDtypeStruct((N, H), jnp.float32),
            jax.ShapeDtypeStruct((N, H), jnp.float32),
        ],
    )(x, Wk, bk.reshape(1, H), Wq, bq.reshape(1, H))


def _part_body(x_ref, batch_ref, u_ref, wx1_ref, wx3_ref, bx_ref, o_ref):
    xb = x_ref[...]
    b = batch_ref[0, 0, :]
    onehot = (b[:, None] == lax.broadcasted_iota(jnp.int32, (1, G), 1)
              ).astype(jnp.float32)
    uw = jnp.dot(u_ref[...], wx3_ref[...], preferred_element_type=jnp.float32)
    o = jnp.dot(xb, wx1_ref[...], preferred_element_type=jnp.float32)
    o = o + jnp.dot(onehot, uw, preferred_element_type=jnp.float32)
    o_ref[...] = o + bx_ref[...]


def _partial(x, batch3d, u, Wx1, Wx3, bx):
    return pl.pallas_call(
        _part_body,
        grid=(TGRID,),
        in_specs=[
            pl.BlockSpec((TB, F_X), lambda i: (i, 0)),
            pl.BlockSpec((1, 1, TB), lambda i: (i, 0, 0)),
            pl.BlockSpec((G, F_U), lambda i: (0, 0)),
            pl.BlockSpec((F_X, F_X), lambda i: (0, 0)),
            pl.BlockSpec((F_U, F_X), lambda i: (0, 0)),
            pl.BlockSpec((1, F_X), lambda i: (0, 0)),
        ],
        out_specs=pl.BlockSpec((TB, F_X), lambda i: (i, 0)),
        out_shape=jax.ShapeDtypeStruct((N, F_X), jnp.float32),
    )(x, batch3d, u, Wx1, Wx3, bx.reshape(1, F_X))


_GDN = lax.GatherDimensionNumbers(
    offset_dims=(), collapsed_slice_dims=(0,), start_index_map=(0,))


def _lane_shuffle(v, idx):
    return lax.gather(v, idx[:, None], dimension_numbers=_GDN,
                      slice_sizes=(1,),
                      mode=lax.GatherScatterMode.PROMISE_IN_BOUNDS)


def _edge_body(k_hbm, q_hbm, src_hbm, dst_hbm, e_hbm, out_hbm,
               sidx, didx, krows, qrows, erows, aerows, zbuf, acc,
               sem_k, sem_q, sem_e, sem_s0, sem_s1):
    sem_s = (sem_s0, sem_s1)
    c = lax.axis_index("c")
    s = lax.axis_index("s")
    wid = s * NC + c
    start = wid * NCHUNK // NW
    end = (wid + 1) * NCHUNK // NW

    def _zrow(i, carry):
        zbuf[i, :] = jnp.zeros((L,), jnp.float32)
        return carry
    lax.fori_loop(0, ROWS_PER_SUB, _zrow, 0)
    pltpu.sync_copy(zbuf, acc.at[pl.ds(s * ROWS_PER_SUB, ROWS_PER_SUB)])

    pltpu.sync_copy(src_hbm.at[pl.ds(start, MAXCH)], sidx)
    pltpu.sync_copy(dst_hbm.at[pl.ds(start, MAXCH)], didx)
    plsc.subcore_barrier()

    EROWS = C * F_E // 128

    def _fire(v, slot):
        pltpu.async_copy(k_hbm.at[sidx.at[v - start]], krows.at[slot], sem_k)
        pltpu.async_copy(q_hbm.at[didx.at[v - start]], qrows.at[slot], sem_q)
        pltpu.async_copy(e_hbm.at[pl.ds(v * EROWS, EROWS)], erows.at[slot],
                         sem_e)

    def _await(v, slot):
        pltpu.make_async_copy(k_hbm.at[sidx.at[v - start]], krows.at[slot],
                              sem_k).wait()
        pltpu.make_async_copy(q_hbm.at[didx.at[v - start]], qrows.at[slot],
                              sem_q).wait()
        pltpu.make_async_copy(e_hbm.at[pl.ds(v * EROWS, EROWS)],
                              erows.at[slot], sem_e).wait()

    lanes = lax.iota(jnp.int32, L)
    msk8 = lanes < 8
    msk4 = (lanes & 4) == 0
    msk2 = (lanes & 2) == 0
    msk1 = (lanes & 1) == 0
    REV = (0, 8, 4, 12, 2, 10, 6, 14, 1, 9, 5, 13, 3, 11, 7, 15)

    def _fold(v, k):
        return v + _lane_shuffle(v, lanes ^ k)

    def _compute(v, slot):
        @pl.when(v - start >= 2)
        def _():
            pltpu.make_async_copy(aerows.at[slot],
                                  acc.at[didx.at[v - 2 - start]],
                                  sem_s[slot]).wait()
        _await(v, slot)
        kr = krows.at[slot]
        qr = qrows.at[slot]
        er = erows.at[slot]
        ar = aerows.at[slot]

        def _dot(i):
            d = kr[i, 0:L] * qr[i, 0:L]
            for g in range(1, H // L):
                d = d + kr[i, g * L:(g + 1) * L] * qr[i, g * L:(g + 1) * L]
            return d

        def _blk16(b, ecarry):
            base = b * 16
            m = []
            for p in range(8):
                d0 = _dot(base + 2 * p)
                d1 = _dot(base + 2 * p + 1)
                m.append(jnp.where(msk8, _fold(d0, 8), _fold(d1, 8)))
            n = [jnp.where(msk4, _fold(m[2 * q], 4), _fold(m[2 * q + 1], 4))
                 for q in range(4)]
            r0 = jnp.where(msk2, _fold(n[0], 2), _fold(n[1], 2))
            r1 = jnp.where(msk2, _fold(n[2], 2), _fold(n[3], 2))
            f = jnp.where(msk1, _fold(r0, 1), _fold(r1, 1))
            a = 1.0 / (1.0 + jnp.exp(-f))
            for j in range(16):
                i = base + j
                aj = _lane_shuffle(a, jnp.full((L,), REV[j], jnp.int32))
                ar[i, :] = aj * er[i // 8, pl.ds((i % 8) * L, L)]
            return ecarry
        lax.fori_loop(0, C // 16, _blk16, 0)
        pltpu.async_copy(ar, acc.at[didx.at[v - start]], sem_s[slot],
                         add=True)

    _fire(start, 0)
    _fire(start + 1, 1)

    def _pair(i2, carry):
        v0 = start + 2 * i2
        v1 = v0 + 1

        @pl.when(v0 < end)
        def _():
            _compute(v0, 0)

            @pl.when(v0 + 2 < end)
            def _():
                _fire(v0 + 2, 0)

        @pl.when(v1 < end)
        def _():
            _compute(v1, 1)

            @pl.when(v1 + 2 < end)
            def _():
                _fire(v1 + 2, 1)
        return carry
    lax.fori_loop(0, MAXCH // 2, _pair, 0)

    pltpu.make_async_copy(aerows.at[0], acc.at[didx.at[0]], sem_s[0]).wait()
    pltpu.make_async_copy(aerows.at[1], acc.at[didx.at[0]], sem_s[1]).wait()

    plsc.subcore_barrier()
    pltpu.sync_copy(acc.at[pl.ds(s * ROWS_PER_SUB, ROWS_PER_SUB)],
                    out_hbm.at[c, pl.ds(s * ROWS_PER_SUB, ROWS_PER_SUB)])


def _edge_aggregate(k_all, q_all, src, dst, e):
    mesh = plsc.VectorSubcoreMesh(core_axis_name="c", subcore_axis_name="s")
    fn = functools.partial(
        pl.kernel,
        mesh=mesh,
        compiler_params=pltpu.CompilerParams(use_tc_tiling_on_sc=False),
        out_type=jax.ShapeDtypeStruct((NC, NPAD, F_E), jnp.float32),
        scratch_types=[
            pltpu.VMEM((MAXCH, C), jnp.int32),
            pltpu.VMEM((MAXCH, C), jnp.int32),
            pltpu.VMEM((2, C, H), jnp.float32),
            pltpu.VMEM((2, C, H), jnp.float32),
            pltpu.VMEM((2, C * F_E // 128, 128), jnp.float32),
            pltpu.VMEM((2, C, F_E), jnp.float32),
            pltpu.VMEM((ROWS_PER_SUB, F_E), jnp.float32),
            pltpu.VMEM_SHARED((NPAD, F_E), jnp.float32),
            pltpu.SemaphoreType.DMA,
            pltpu.SemaphoreType.DMA,
            pltpu.SemaphoreType.DMA,
            pltpu.SemaphoreType.DMA,
            pltpu.SemaphoreType.DMA,
        ],
    )(_edge_body)
    return fn(k_all, q_all, src.reshape(NCHUNK, C), dst.reshape(NCHUNK, C),
              e.reshape(E * F_E // 128, 128))


def _final_body(p_ref, a0_ref, a1_ref, wx2_ref, o_ref):
    aggb = a0_ref[...] + a1_ref[...]
    o_ref[...] = p_ref[...] + jnp.dot(aggb, wx2_ref[...],
                                      preferred_element_type=jnp.float32)


def _final(part, agg0, agg1, Wx2):
    return pl.pallas_call(
        _final_body,
        grid=(TGRID,),
        in_specs=[
            pl.BlockSpec((TB, F_X), lambda i: (i, 0)),
            pl.BlockSpec((TB, F_E), lambda i: (i, 0)),
            pl.BlockSpec((TB, F_E), lambda i: (i, 0)),
            pl.BlockSpec((F_E, F_X), lambda i: (0, 0)),
        ],
        out_specs=pl.BlockSpec((TB, F_X), lambda i: (i, 0)),
        out_shape=jax.ShapeDtypeStruct((N, F_X), jnp.float32),
    )(part, agg0, agg1, Wx2)


def kernel(x, edge_index, e, u, batch, Wk, bk, Wq, bq, Wx, bx):
    src = edge_index[0].astype(jnp.int32)
    dst = edge_index[1].astype(jnp.int32)
    batch3d = batch.astype(jnp.int32).reshape(TGRID, 1, TB)

    k_all, q_all = _project(x, Wk, bk, Wq, bq)
    agg2 = _edge_aggregate(k_all, q_all, src, dst, e)
    part = _partial(x, batch3d, u, Wx[:F_X], Wx[F_X + F_E:], bx)
    return _final(part, agg2[0, :N], agg2[1, :N], Wx[F_X:F_X + F_E])


# speedup vs baseline: 1.0409x; 1.0016x over previous
import functools

import jax
import jax.numpy as jnp
from jax import lax
from jax.experimental import pallas as pl
from jax.experimental.pallas import tpu as pltpu
from jax.experimental.pallas import tpu_sc as plsc

N = 10000
E = 160000
F_X = 256
F_E = 16
F_U = 64
H = 128
G = 16

NC = 2
NS = 16
L = 16
NW = NC * NS

C = 128
NCHUNK = E // C
MAXCH = (NCHUNK + NW - 1) // NW
NPAD = 10240
ROWS_PER_SUB = NPAD // NS

TB = 1000
TGRID = N // TB


def _proj_body(x_ref, wk_ref, bk_ref, wq_ref, bq_ref, k_ref, q_ref):
    xb = x_ref[...]
    kv = jnp.dot(xb, wk_ref[...], preferred_element_type=jnp.float32) + bk_ref[...]
    qv = jnp.dot(xb, wq_ref[...], preferred_element_type=jnp.float32) + bq_ref[...]
    k_ref[...] = kv
    q_ref[...] = qv


def _project(x, Wk, bk, Wq, bq):
    return pl.pallas_call(
        _proj_body,
        grid=(TGRID,),
        in_specs=[
            pl.BlockSpec((TB, F_X), lambda i: (i, 0)),
            pl.BlockSpec((F_X, H), lambda i: (0, 0)),
            pl.BlockSpec((1, H), lambda i: (0, 0)),
            pl.BlockSpec((F_X, H), lambda i: (0, 0)),
            pl.BlockSpec((1, H), lambda i: (0, 0)),
        ],
        out_specs=[
            pl.BlockSpec((TB, H), lambda i: (i, 0)),
            pl.BlockSpec((TB, H), lambda i: (i, 0)),
        ],
        out_shape=[
            jax.ShapeDtypeStruct((N, H), jnp.float32),
            jax.ShapeDtypeStruct((N, H), jnp.float32),
        ],
    )(x, Wk, bk.reshape(1, H), Wq, bq.reshape(1, H))


def _part_body(x_ref, batch_ref, u_ref, wx1_ref, wx3_ref, bx_ref, o_ref):
    xb = x_ref[...]
    b = batch_ref[0, 0, :]
    onehot = (b[:, None] == lax.broadcasted_iota(jnp.int32, (1, G), 1)
              ).astype(jnp.float32)
    uw = jnp.dot(u_ref[...], wx3_ref[...], preferred_element_type=jnp.float32)
    o = jnp.dot(xb, wx1_ref[...], preferred_element_type=jnp.float32)
    o = o + jnp.dot(onehot, uw, preferred_element_type=jnp.float32)
    o_ref[...] = o + bx_ref[...]


def _partial(x, batch3d, u, Wx1, Wx3, bx):
    return pl.pallas_call(
        _part_body,
        grid=(TGRID,),
        in_specs=[
            pl.BlockSpec((TB, F_X), lambda i: (i, 0)),
            pl.BlockSpec((1, 1, TB), lambda i: (i, 0, 0)),
            pl.BlockSpec((G, F_U), lambda i: (0, 0)),
            pl.BlockSpec((F_X, F_X), lambda i: (0, 0)),
            pl.BlockSpec((F_U, F_X), lambda i: (0, 0)),
            pl.BlockSpec((1, F_X), lambda i: (0, 0)),
        ],
        out_specs=pl.BlockSpec((TB, F_X), lambda i: (i, 0)),
        out_shape=jax.ShapeDtypeStruct((N, F_X), jnp.float32),
    )(x, batch3d, u, Wx1, Wx3, bx.reshape(1, F_X))


_GDN = lax.GatherDimensionNumbers(
    offset_dims=(), collapsed_slice_dims=(0,), start_index_map=(0,))


def _lane_shuffle(v, idx):
    return lax.gather(v, idx[:, None], dimension_numbers=_GDN,
                      slice_sizes=(1,),
                      mode=lax.GatherScatterMode.PROMISE_IN_BOUNDS)


def _edge_body(k_hbm, q_hbm, src_hbm, dst_hbm, e_hbm, out_hbm,
               sidx, didx, krows, qrows, erows, aerows, zbuf, acc,
               sem_k, sem_q, sem_e, sem_s0, sem_s1):
    sem_s = (sem_s0, sem_s1)
    c = lax.axis_index("c")
    s = lax.axis_index("s")
    wid = s * NC + c
    start = wid * NCHUNK // NW
    end = (wid + 1) * NCHUNK // NW

    def _zrow(i, carry):
        zbuf[i, :] = jnp.zeros((L,), jnp.float32)
        return carry
    lax.fori_loop(0, ROWS_PER_SUB, _zrow, 0)
    pltpu.sync_copy(zbuf, acc.at[pl.ds(s * ROWS_PER_SUB, ROWS_PER_SUB)])

    pltpu.sync_copy(src_hbm.at[pl.ds(start, MAXCH)], sidx)
    pltpu.sync_copy(dst_hbm.at[pl.ds(start, MAXCH)], didx)
    plsc.subcore_barrier()

    def _fire(v, slot):
        pltpu.async_copy(k_hbm.at[sidx.at[v - start]], krows.at[slot], sem_k)
        pltpu.async_copy(q_hbm.at[didx.at[v - start]], qrows.at[slot], sem_q)
        pltpu.async_copy(e_hbm.at[pl.ds(v * C, C)], erows.at[slot], sem_e)

    def _await(v, slot):
        pltpu.make_async_copy(k_hbm.at[sidx.at[v - start]], krows.at[slot],
                              sem_k).wait()
        pltpu.make_async_copy(q_hbm.at[didx.at[v - start]], qrows.at[slot],
                              sem_q).wait()
        pltpu.make_async_copy(e_hbm.at[pl.ds(v * C, C)],
                              erows.at[slot], sem_e).wait()

    lanes = lax.iota(jnp.int32, L)
    msk8 = lanes < 8
    msk4 = (lanes & 4) == 0
    msk2 = (lanes & 2) == 0
    msk1 = (lanes & 1) == 0
    REV = (0, 8, 4, 12, 2, 10, 6, 14, 1, 9, 5, 13, 3, 11, 7, 15)

    def _fold(v, k):
        return v + _lane_shuffle(v, lanes ^ k)

    def _compute(v, slot):
        @pl.when(v - start >= 2)
        def _():
            pltpu.make_async_copy(aerows.at[slot],
                                  acc.at[didx.at[v - 2 - start]],
                                  sem_s[slot]).wait()
        _await(v, slot)
        kr = krows.at[slot]
        qr = qrows.at[slot]
        er = erows.at[slot]
        ar = aerows.at[slot]

        def _dot(i):
            d = kr[i, 0:L] * qr[i, 0:L]
            for g in range(1, H // L):
                d = d + kr[i, g * L:(g + 1) * L] * qr[i, g * L:(g + 1) * L]
            return d

        def _blk16(b, ecarry):
            base = b * 16
            m = []
            for p in range(8):
                d0 = _dot(base + 2 * p)
                d1 = _dot(base + 2 * p + 1)
                m.append(jnp.where(msk8, _fold(d0, 8), _fold(d1, 8)))
            n = [jnp.where(msk4, _fold(m[2 * q], 4), _fold(m[2 * q + 1], 4))
                 for q in range(4)]
            r0 = jnp.where(msk2, _fold(n[0], 2), _fold(n[1], 2))
            r1 = jnp.where(msk2, _fold(n[2], 2), _fold(n[3], 2))
            f = jnp.where(msk1, _fold(r0, 1), _fold(r1, 1))
            a = 1.0 / (1.0 + jnp.exp(-f))
            for j in range(16):
                i = base + j
                aj = _lane_shuffle(a, jnp.full((L,), REV[j], jnp.int32))
                ar[i, :] = aj * er[i, :]
            return ecarry
        lax.fori_loop(0, C // 16, _blk16, 0)
        pltpu.async_copy(ar, acc.at[didx.at[v - start]], sem_s[slot],
                         add=True)

    _fire(start, 0)
    _fire(start + 1, 1)

    def _pair(i2, carry):
        v0 = start + 2 * i2
        v1 = v0 + 1

        @pl.when(v0 < end)
        def _():
            _compute(v0, 0)

            @pl.when(v0 + 2 < end)
            def _():
                _fire(v0 + 2, 0)

        @pl.when(v1 < end)
        def _():
            _compute(v1, 1)

            @pl.when(v1 + 2 < end)
            def _():
                _fire(v1 + 2, 1)
        return carry
    lax.fori_loop(0, MAXCH // 2, _pair, 0)

    pltpu.make_async_copy(aerows.at[0], acc.at[didx.at[0]], sem_s[0]).wait()
    pltpu.make_async_copy(aerows.at[1], acc.at[didx.at[0]], sem_s[1]).wait()

    plsc.subcore_barrier()
    pltpu.sync_copy(acc.at[pl.ds(s * ROWS_PER_SUB, ROWS_PER_SUB)],
                    out_hbm.at[c, pl.ds(s * ROWS_PER_SUB, ROWS_PER_SUB)])


def _edge_aggregate(k_all, q_all, src, dst, e):
    mesh = plsc.VectorSubcoreMesh(core_axis_name="c", subcore_axis_name="s")
    fn = functools.partial(
        pl.kernel,
        mesh=mesh,
        compiler_params=pltpu.CompilerParams(use_tc_tiling_on_sc=False),
        out_type=jax.ShapeDtypeStruct((NC, NPAD, F_E), jnp.float32),
        scratch_types=[
            pltpu.VMEM((MAXCH, C), jnp.int32),
            pltpu.VMEM((MAXCH, C), jnp.int32),
            pltpu.VMEM((2, C, H), jnp.float32),
            pltpu.VMEM((2, C, H), jnp.float32),
            pltpu.VMEM((2, C, F_E), jnp.float32),
            pltpu.VMEM((2, C, F_E), jnp.float32),
            pltpu.VMEM((ROWS_PER_SUB, F_E), jnp.float32),
            pltpu.VMEM_SHARED((NPAD, F_E), jnp.float32),
            pltpu.SemaphoreType.DMA,
            pltpu.SemaphoreType.DMA,
            pltpu.SemaphoreType.DMA,
            pltpu.SemaphoreType.DMA,
            pltpu.SemaphoreType.DMA,
        ],
    )(_edge_body)
    return fn(k_all, q_all, src.reshape(NCHUNK, C), dst.reshape(NCHUNK, C), e)


def _final_body(p_ref, a0_ref, a1_ref, wx2_ref, o_ref):
    aggb = a0_ref[...] + a1_ref[...]
    o_ref[...] = p_ref[...] + jnp.dot(aggb, wx2_ref[...],
                                      preferred_element_type=jnp.float32)


def _final(part, agg0, agg1, Wx2):
    return pl.pallas_call(
        _final_body,
        grid=(TGRID,),
        in_specs=[
            pl.BlockSpec((TB, F_X), lambda i: (i, 0)),
            pl.BlockSpec((TB, F_E), lambda i: (i, 0)),
            pl.BlockSpec((TB, F_E), lambda i: (i, 0)),
            pl.BlockSpec((F_E, F_X), lambda i: (0, 0)),
        ],
        out_specs=pl.BlockSpec((TB, F_X), lambda i: (i, 0)),
        out_shape=jax.ShapeDtypeStruct((N, F_X), jnp.float32),
    )(part, agg0, agg1, Wx2)


def kernel(x, edge_index, e, u, batch, Wk, bk, Wq, bq, Wx, bx):
    src = edge_index[0].astype(jnp.int32)
    dst = edge_index[1].astype(jnp.int32)
    batch3d = batch.astype(jnp.int32).reshape(TGRID, 1, TB)

    k_all, q_all = _project(x, Wk, bk, Wq, bq)
    agg2 = _edge_aggregate(k_all, q_all, src, dst, e)
    part = _partial(x, batch3d, u, Wx[:F_X], Wx[F_X + F_E:], bx)
    return _final(part, agg2[0, :N], agg2[1, :N], Wx[F_X:F_X + F_E])
